# baseline (speedup 1.0000x reference)
import functools

import jax
import jax.numpy as jnp
from jax import lax
from jax.experimental import pallas as pl
from jax.experimental.pallas import tpu as pltpu

B, C, H, W = 8, 3, 224, 224
ZC = 256
E = 256
K = 1024
DS = 8
HP = H // DS
N = B * HP * HP
P = C * DS * DS
TN = 896
NT = N // TN


def _main_body(patch_ref, w1_ref, encb_ref, prewt_ref, preb_ref, cbt_ref,
               cnorm_ref, z_ref, tok_ref):
    p = patch_ref[...]
    ze = jnp.dot(p, w1_ref[...],
                 preferred_element_type=jnp.float32) + encb_ref[...]
    z = jnp.dot(ze.astype(jnp.bfloat16), prewt_ref[...],
                preferred_element_type=jnp.float32) + preb_ref[...]
    z_ref[...] = z
    mm = jnp.dot(z.astype(jnp.bfloat16), cbt_ref[...],
                 preferred_element_type=jnp.float32)
    rnorm = jnp.sum(z * z, axis=1, keepdims=True)
    dist = (rnorm + cnorm_ref[...]) - 2.0 * mm
    m = jnp.min(dist, axis=1, keepdims=True)
    iota = lax.broadcasted_iota(jnp.int32, dist.shape, 1)
    cand = jnp.where(dist == m, iota, K)
    tok_ref[0, 0, :] = jnp.min(cand, axis=1)


def _precompute_body(cb_ref, postwt_ref, wd2_ref, postb_ref, decb_ref,
                     cb2_ref, cnorm_ref):
    wpd = jnp.dot(postwt_ref[...], wd2_ref[...],
                  preferred_element_type=jnp.float32,
                  precision=lax.Precision.HIGHEST)
    vb = jnp.dot(postb_ref[...], wd2_ref[...],
                 preferred_element_type=jnp.float32,
                 precision=lax.Precision.HIGHEST) + decb_ref[...]
    cb = cb_ref[...]
    cb2_ref[...] = jnp.dot(cb, wpd, preferred_element_type=jnp.float32,
                           precision=lax.Precision.HIGHEST) + vb
    cnorm_ref[0, :] = jnp.sum(cb * cb, axis=1)


def kernel(x, enc_W, enc_b, pre_W, pre_b, codebook, post_W, post_b, dec_W, dec_b):
    patches = x.reshape(B, C, HP, DS, HP, DS).transpose(0, 2, 4, 1, 3, 5)
    patches = patches.reshape(N, P).astype(jnp.bfloat16)
    w1 = enc_W.reshape(ZC, P).T.astype(jnp.bfloat16)
    prewt = pre_W.T.astype(jnp.bfloat16)
    cbt = codebook.T.astype(jnp.bfloat16)
    wd2 = dec_W[:, :, ::-1, ::-1].transpose(1, 0, 2, 3).reshape(ZC, P)
    postwt = post_W.T
    encb2 = enc_b.reshape(1, ZC)
    preb2 = pre_b.reshape(1, E)
    postb2 = post_b.reshape(1, ZC)
    decb2 = jnp.repeat(dec_b, DS * DS).reshape(1, P)

    cb2, cnorm = pl.pallas_call(
        _precompute_body,
        out_shape=(
            jax.ShapeDtypeStruct((K, P), jnp.float32),
            jax.ShapeDtypeStruct((1, K), jnp.float32),
        ),
    )(codebook, postwt, wd2, postb2, decb2)

    z_flat, tok3 = pl.pallas_call(
        _main_body,
        grid=(NT,),
        in_specs=[
            pl.BlockSpec((TN, P), lambda i: (i, 0)),
            pl.BlockSpec((P, ZC), lambda i: (0, 0)),
            pl.BlockSpec((1, ZC), lambda i: (0, 0)),
            pl.BlockSpec((ZC, E), lambda i: (0, 0)),
            pl.BlockSpec((1, E), lambda i: (0, 0)),
            pl.BlockSpec((E, K), lambda i: (0, 0)),
            pl.BlockSpec((1, K), lambda i: (0, 0)),
        ],
        out_specs=(
            pl.BlockSpec((TN, E), lambda i: (i, 0)),
            pl.BlockSpec((1, 1, TN), lambda i: (i, 0, 0)),
        ),
        out_shape=(
            jax.ShapeDtypeStruct((N, E), jnp.float32),
            jax.ShapeDtypeStruct((NT, 1, TN), jnp.int32),
        ),
    )(patches, w1, encb2, prewt, preb2, cbt, cnorm)

    tokens = tok3.reshape(N)

    zq_flat = jnp.take(codebook, tokens, axis=0)
    recp = jnp.take(cb2, tokens, axis=0)

    z = z_flat.reshape(B, HP, HP, E).transpose(0, 3, 1, 2)
    z_q = zq_flat.reshape(B, HP, HP, E).transpose(0, 3, 1, 2)
    rec = recp.reshape(B, HP, HP, C, DS, DS).transpose(0, 3, 1, 4, 2, 5)
    rec = rec.reshape(B, C, H, W)
    return (z, z_q, rec)


# speedup vs baseline: 1.1560x; 1.1560x over previous
import functools

import jax
import jax.numpy as jnp
from jax import lax
from jax.experimental import pallas as pl
from jax.experimental.pallas import tpu as pltpu

B, C, H, W = 8, 3, 224, 224
ZC = 256
E = 256
K = 1024
DS = 8
HP = H // DS
N = B * HP * HP
P = C * DS * DS
G = E + P
TN = 896
NT = N // TN


def _main_body(patch_ref, w1_ref, encb_ref, prewt_ref, preb_ref, cbt_ref,
               cnorm_ref, gt_ref, z_ref, zq_ref, recp_ref):
    p = patch_ref[...]
    ze = jnp.dot(p, w1_ref[...],
                 preferred_element_type=jnp.float32) + encb_ref[...]
    z = jnp.dot(ze.astype(jnp.bfloat16), prewt_ref[...],
                preferred_element_type=jnp.float32) + preb_ref[...]
    z_ref[...] = z
    mm = jnp.dot(z.astype(jnp.bfloat16), cbt_ref[...],
                 preferred_element_type=jnp.float32)
    rnorm = jnp.sum(z * z, axis=1, keepdims=True)
    dist = (rnorm + cnorm_ref[...]) - 2.0 * mm
    m = jnp.min(dist, axis=1, keepdims=True)
    iota = lax.broadcasted_iota(jnp.int32, dist.shape, 1)
    cand = jnp.where(dist == m, iota, K)
    tok = jnp.min(cand, axis=1, keepdims=True)
    onehot = (iota == tok).astype(jnp.bfloat16)
    g = jnp.dot(onehot, gt_ref[...],
                preferred_element_type=jnp.float32)
    zq_ref[...] = g[:, :E]
    recp_ref[...] = g[:, E:]


def _precompute_body(cb_ref, postwt_ref, wd2_ref, postb_ref, decb_ref,
                     cnorm_ref, gt_ref):
    wpd = jnp.dot(postwt_ref[...], wd2_ref[...],
                  preferred_element_type=jnp.float32,
                  precision=lax.Precision.HIGHEST)
    vb = jnp.dot(postb_ref[...], wd2_ref[...],
                 preferred_element_type=jnp.float32,
                 precision=lax.Precision.HIGHEST) + decb_ref[...]
    cb = cb_ref[...]
    cb2 = jnp.dot(cb, wpd, preferred_element_type=jnp.float32,
                  precision=lax.Precision.HIGHEST) + vb
    cnorm_ref[0, :] = jnp.sum(cb * cb, axis=1)
    gt_ref[:, :E] = cb.astype(jnp.bfloat16)
    gt_ref[:, E:] = cb2.astype(jnp.bfloat16)


def kernel(x, enc_W, enc_b, pre_W, pre_b, codebook, post_W, post_b, dec_W, dec_b):
    patches = x.reshape(B, C, HP, DS, HP, DS).transpose(0, 2, 4, 1, 3, 5)
    patches = patches.reshape(N, P).astype(jnp.bfloat16)
    w1 = enc_W.reshape(ZC, P).T.astype(jnp.bfloat16)
    prewt = pre_W.T.astype(jnp.bfloat16)
    cbt = codebook.T.astype(jnp.bfloat16)
    wd2 = dec_W[:, :, ::-1, ::-1].transpose(1, 0, 2, 3).reshape(ZC, P)
    postwt = post_W.T
    encb2 = enc_b.reshape(1, ZC)
    preb2 = pre_b.reshape(1, E)
    postb2 = post_b.reshape(1, ZC)
    decb2 = jnp.repeat(dec_b, DS * DS).reshape(1, P)

    cnorm, gtable = pl.pallas_call(
        _precompute_body,
        out_shape=(
            jax.ShapeDtypeStruct((1, K), jnp.float32),
            jax.ShapeDtypeStruct((K, G), jnp.bfloat16),
        ),
    )(codebook, postwt, wd2, postb2, decb2)

    z_flat, zq_flat, recp = pl.pallas_call(
        _main_body,
        grid=(NT,),
        in_specs=[
            pl.BlockSpec((TN, P), lambda i: (i, 0)),
            pl.BlockSpec((P, ZC), lambda i: (0, 0)),
            pl.BlockSpec((1, ZC), lambda i: (0, 0)),
            pl.BlockSpec((ZC, E), lambda i: (0, 0)),
            pl.BlockSpec((1, E), lambda i: (0, 0)),
            pl.BlockSpec((E, K), lambda i: (0, 0)),
            pl.BlockSpec((1, K), lambda i: (0, 0)),
            pl.BlockSpec((K, G), lambda i: (0, 0)),
        ],
        out_specs=(
            pl.BlockSpec((TN, E), lambda i: (i, 0)),
            pl.BlockSpec((TN, E), lambda i: (i, 0)),
            pl.BlockSpec((TN, P), lambda i: (i, 0)),
        ),
        out_shape=(
            jax.ShapeDtypeStruct((N, E), jnp.float32),
            jax.ShapeDtypeStruct((N, E), jnp.float32),
            jax.ShapeDtypeStruct((N, P), jnp.float32),
        ),
    )(patches, w1, encb2, prewt, preb2, cbt, cnorm, gtable)

    z = z_flat.reshape(B, HP, HP, E).transpose(0, 3, 1, 2)
    z_q = zq_flat.reshape(B, HP, HP, E).transpose(0, 3, 1, 2)
    rec = recp.reshape(B, HP, HP, C, DS, DS).transpose(0, 3, 1, 4, 2, 5)
    rec = rec.reshape(B, C, H, W)
    return (z, z_q, rec)


# speedup vs baseline: 1.1742x; 1.0157x over previous
import functools

import jax
import jax.numpy as jnp
from jax import lax
from jax.experimental import pallas as pl
from jax.experimental.pallas import tpu as pltpu

B, C, H, W = 8, 3, 224, 224
ZC = 256
E = 256
K = 1024
DS = 8
HP = H // DS
N = B * HP * HP
P = C * DS * DS
G = E + P
TN = HP * HP


def _main_body(patch_ref, w1_ref, encb_ref, prewt_ref, preb_ref, cbt_ref,
               cnorm_ref, gt_ref, z_ref, zq_ref, recp_ref):
    p = patch_ref[...]
    ze = jnp.dot(p, w1_ref[...],
                 preferred_element_type=jnp.float32) + encb_ref[...]
    z = jnp.dot(ze.astype(jnp.bfloat16), prewt_ref[...],
                preferred_element_type=jnp.float32) + preb_ref[...]
    z_ref[0] = z.T
    mm = jnp.dot(z.astype(jnp.bfloat16), cbt_ref[...],
                 preferred_element_type=jnp.float32)
    rnorm = jnp.sum(z * z, axis=1, keepdims=True)
    dist = (rnorm + cnorm_ref[...]) - 2.0 * mm
    m = jnp.min(dist, axis=1, keepdims=True)
    iota = lax.broadcasted_iota(jnp.int32, dist.shape, 1)
    cand = jnp.where(dist == m, iota, K)
    tok = jnp.min(cand, axis=1, keepdims=True)
    onehot = (iota == tok).astype(jnp.bfloat16)
    g = jnp.dot(onehot, gt_ref[...],
                preferred_element_type=jnp.float32)
    gt = g.T
    zq_ref[0] = gt[:E]
    recp_ref[0] = gt[E:]


def _precompute_body(cb_ref, postwt_ref, wd2_ref, postb_ref, decb_ref,
                     cnorm_ref, gt_ref):
    wpd = jnp.dot(postwt_ref[...], wd2_ref[...],
                  preferred_element_type=jnp.float32,
                  precision=lax.Precision.HIGHEST)
    vb = jnp.dot(postb_ref[...], wd2_ref[...],
                 preferred_element_type=jnp.float32,
                 precision=lax.Precision.HIGHEST) + decb_ref[...]
    cb = cb_ref[...]
    cb2 = jnp.dot(cb, wpd, preferred_element_type=jnp.float32,
                  precision=lax.Precision.HIGHEST) + vb
    cnorm_ref[0, :] = jnp.sum(cb * cb, axis=1)
    gt_ref[:, :E] = cb.astype(jnp.bfloat16)
    gt_ref[:, E:] = cb2.astype(jnp.bfloat16)


def kernel(x, enc_W, enc_b, pre_W, pre_b, codebook, post_W, post_b, dec_W, dec_b):
    patches = x.reshape(B, C, HP, DS, HP, DS).transpose(0, 2, 4, 1, 3, 5)
    patches = patches.reshape(N, P).astype(jnp.bfloat16)
    w1 = enc_W.reshape(ZC, P).T.astype(jnp.bfloat16)
    prewt = pre_W.T.astype(jnp.bfloat16)
    cbt = codebook.T.astype(jnp.bfloat16)
    wd2 = dec_W[:, :, ::-1, ::-1].transpose(1, 0, 2, 3).reshape(ZC, P)
    postwt = post_W.T
    encb2 = enc_b.reshape(1, ZC)
    preb2 = pre_b.reshape(1, E)
    postb2 = post_b.reshape(1, ZC)
    decb2 = jnp.repeat(dec_b, DS * DS).reshape(1, P)

    cnorm, gtable = pl.pallas_call(
        _precompute_body,
        out_shape=(
            jax.ShapeDtypeStruct((1, K), jnp.float32),
            jax.ShapeDtypeStruct((K, G), jnp.bfloat16),
        ),
    )(codebook, postwt, wd2, postb2, decb2)

    zt, zqt, rect = pl.pallas_call(
        _main_body,
        grid=(B,),
        in_specs=[
            pl.BlockSpec((TN, P), lambda i: (i, 0)),
            pl.BlockSpec((P, ZC), lambda i: (0, 0)),
            pl.BlockSpec((1, ZC), lambda i: (0, 0)),
            pl.BlockSpec((ZC, E), lambda i: (0, 0)),
            pl.BlockSpec((1, E), lambda i: (0, 0)),
            pl.BlockSpec((E, K), lambda i: (0, 0)),
            pl.BlockSpec((1, K), lambda i: (0, 0)),
            pl.BlockSpec((K, G), lambda i: (0, 0)),
        ],
        out_specs=(
            pl.BlockSpec((1, E, TN), lambda i: (i, 0, 0)),
            pl.BlockSpec((1, E, TN), lambda i: (i, 0, 0)),
            pl.BlockSpec((1, P, TN), lambda i: (i, 0, 0)),
        ),
        out_shape=(
            jax.ShapeDtypeStruct((B, E, TN), jnp.float32),
            jax.ShapeDtypeStruct((B, E, TN), jnp.float32),
            jax.ShapeDtypeStruct((B, P, TN), jnp.float32),
        ),
    )(patches, w1, encb2, prewt, preb2, cbt, cnorm, gtable)

    z = zt.reshape(B, E, HP, HP)
    z_q = zqt.reshape(B, E, HP, HP)
    rec = rect.reshape(B, C, DS, DS, HP, HP).transpose(0, 1, 4, 2, 5, 3)
    rec = rec.reshape(B, C, H, W)
    return (z, z_q, rec)


# speedup vs baseline: 1.3198x; 1.1240x over previous
import functools

import jax
import jax.numpy as jnp
from jax import lax
from jax.experimental import pallas as pl
from jax.experimental.pallas import tpu as pltpu
from jax.experimental.pallas import tpu_sc as plsc

B, C, H, W = 8, 3, 224, 224
ZC = 256
E = 256
K = 1024
DS = 8
HP = H // DS
N = B * HP * HP
P = C * DS * DS
TN = HP * HP
PW = 256

NSC = 2
NTEC = 16
NWORK = NSC * NTEC
QI = HP // 4
QTOK = QI * HP
QPAD = 208
QY = QI * DS


def _main_body(patch_ref, w1_ref, encb_ref, prewt_ref, preb_ref, cbt_ref,
               cnorm_ref, gt_ref, z_ref, zq_ref, tok_ref):
    p = patch_ref[...]
    ze = jnp.dot(p, w1_ref[...],
                 preferred_element_type=jnp.float32) + encb_ref[...]
    z = jnp.dot(ze.astype(jnp.bfloat16), prewt_ref[...],
                preferred_element_type=jnp.float32) + preb_ref[...]
    z_ref[0] = z.T
    mm = jnp.dot(z.astype(jnp.bfloat16), cbt_ref[...],
                 preferred_element_type=jnp.float32)
    rnorm = jnp.sum(z * z, axis=1, keepdims=True)
    dist = (rnorm + cnorm_ref[...]) - 2.0 * mm
    m = jnp.min(dist, axis=1, keepdims=True)
    iota = lax.broadcasted_iota(jnp.int32, dist.shape, 1)
    cand = jnp.where(dist == m, iota, K)
    tok = jnp.min(cand, axis=1, keepdims=True)
    tok_ref[0, 0, :] = tok[:, 0]
    onehot = (iota == tok).astype(jnp.bfloat16)
    g = jnp.dot(onehot, gt_ref[...],
                preferred_element_type=jnp.float32)
    zq_ref[0] = g.T


def _precompute_body(cb_ref, postwt_ref, wd2_ref, postb_ref, decb_ref,
                     cnorm_ref, gt_ref, cb2_ref):
    wpd = jnp.dot(postwt_ref[...], wd2_ref[...],
                  preferred_element_type=jnp.float32,
                  precision=lax.Precision.HIGHEST)
    vb = jnp.dot(postb_ref[...], wd2_ref[...],
                 preferred_element_type=jnp.float32,
                 precision=lax.Precision.HIGHEST) + decb_ref[...]
    cb = cb_ref[...]
    cb2_ref[:, :P] = jnp.dot(cb, wpd, preferred_element_type=jnp.float32,
                             precision=lax.Precision.HIGHEST) + vb
    cb2_ref[:, P:] = jnp.zeros((K, PW - P), jnp.float32)
    cnorm_ref[0, :] = jnp.sum(cb * cb, axis=1)
    gt_ref[...] = cb.astype(jnp.bfloat16)


def _rec_sc_body(tok_hbm, cb2_hbm, rec_hbm, idx_v, rows_v, out_v, sem):
    wid = lax.axis_index("s") * NSC + lax.axis_index("c")
    b = wid >> 2
    q = wid & 3
    base = b * TN + q * QTOK
    off = base & 7
    base8 = pl.multiple_of(base - off, 8)
    pltpu.sync_copy(tok_hbm.at[pl.ds(base8, QPAD)], idx_v)
    pltpu.async_copy(cb2_hbm.at[idx_v], rows_v, sem).wait()
    lanes = lax.iota(jnp.int32, 16)
    rowpat = lanes >> 3
    colpat = lanes & 7
    for cc in range(C):
        def yy_body(yy, carry):
            il = yy >> 3
            di = yy & 7
            colbase = cc * 64 + di * 8
            rowb = off + il * HP
            cvec = colpat + colbase
            for k in range(14):
                rvec = rowpat + (rowb + 2 * k)
                vals = plsc.load_gather(rows_v, [rvec, cvec])
                out_v[cc, yy, pl.ds(16 * k, 16)] = vals
            return carry
        lax.fori_loop(0, QY, yy_body, 0)
    yb = pl.multiple_of(q * QY, 8)
    for cc in range(C):
        pltpu.sync_copy(out_v.at[cc],
                        rec_hbm.at[b, cc, pl.ds(yb, QY), :])


def _rec_gather(tok_pad, cb2):
    mesh = plsc.VectorSubcoreMesh(core_axis_name="c", subcore_axis_name="s")
    return pl.kernel(
        _rec_sc_body,
        mesh=mesh,
        compiler_params=pltpu.CompilerParams(needs_layout_passes=False),
        out_type=jax.ShapeDtypeStruct((B, C, H, W), jnp.float32),
        scratch_types=[
            pltpu.VMEM((QPAD,), jnp.int32),
            pltpu.VMEM((QPAD, PW), jnp.float32),
            pltpu.VMEM((C, QY, W), jnp.float32),
            pltpu.SemaphoreType.DMA,
        ],
    )(tok_pad, cb2)


def kernel(x, enc_W, enc_b, pre_W, pre_b, codebook, post_W, post_b, dec_W, dec_b):
    patches = x.reshape(B, C, HP, DS, HP, DS).transpose(0, 2, 4, 1, 3, 5)
    patches = patches.reshape(N, P).astype(jnp.bfloat16)
    w1 = enc_W.reshape(ZC, P).T.astype(jnp.bfloat16)
    prewt = pre_W.T.astype(jnp.bfloat16)
    cbt = codebook.T.astype(jnp.bfloat16)
    wd2 = dec_W[:, :, ::-1, ::-1].transpose(1, 0, 2, 3).reshape(ZC, P)
    postwt = post_W.T
    encb2 = enc_b.reshape(1, ZC)
    preb2 = pre_b.reshape(1, E)
    postb2 = post_b.reshape(1, ZC)
    decb2 = jnp.repeat(dec_b, DS * DS).reshape(1, P)

    cnorm, gtable, cb2 = pl.pallas_call(
        _precompute_body,
        out_shape=(
            jax.ShapeDtypeStruct((1, K), jnp.float32),
            jax.ShapeDtypeStruct((K, E), jnp.bfloat16),
            jax.ShapeDtypeStruct((K, PW), jnp.float32),
        ),
    )(codebook, postwt, wd2, postb2, decb2)

    zt, zqt, tok = pl.pallas_call(
        _main_body,
        grid=(B,),
        in_specs=[
            pl.BlockSpec((TN, P), lambda i: (i, 0)),
            pl.BlockSpec((P, ZC), lambda i: (0, 0)),
            pl.BlockSpec((1, ZC), lambda i: (0, 0)),
            pl.BlockSpec((ZC, E), lambda i: (0, 0)),
            pl.BlockSpec((1, E), lambda i: (0, 0)),
            pl.BlockSpec((E, K), lambda i: (0, 0)),
            pl.BlockSpec((1, K), lambda i: (0, 0)),
            pl.BlockSpec((K, E), lambda i: (0, 0)),
        ],
        out_specs=(
            pl.BlockSpec((1, E, TN), lambda i: (i, 0, 0)),
            pl.BlockSpec((1, E, TN), lambda i: (i, 0, 0)),
            pl.BlockSpec((1, 1, TN), lambda i: (i, 0, 0)),
        ),
        out_shape=(
            jax.ShapeDtypeStruct((B, E, TN), jnp.float32),
            jax.ShapeDtypeStruct((B, E, TN), jnp.float32),
            jax.ShapeDtypeStruct((B, 1, TN), jnp.int32),
        ),
    )(patches, w1, encb2, prewt, preb2, cbt, cnorm, gtable)

    tok_pad = jnp.concatenate([tok.reshape(N), jnp.zeros(16, jnp.int32)])
    rec = _rec_gather(tok_pad, cb2)

    z = zt.reshape(B, E, HP, HP)
    z_q = zqt.reshape(B, E, HP, HP)
    return (z, z_q, rec)


# speedup vs baseline: 2.5298x; 1.9168x over previous
import functools

import jax
import jax.numpy as jnp
from jax import lax
from jax.experimental import pallas as pl
from jax.experimental.pallas import tpu as pltpu

B, C, H, W = 8, 3, 224, 224
ZC = 256
E = 256
K = 1024
DS = 8
HP = H // DS
N = B * HP * HP
P = C * DS * DS
G = E + P
TN = HP * HP


def _main_body(x_ref, w1_ref, encb_ref, prewt_ref, preb_ref, cbt_ref,
               cnorm_ref, gt_ref, z_ref, zq_ref, rec_ref):
    xb = x_ref[0]
    x5 = xb.reshape(C, HP, DS, HP, DS)
    p6 = jnp.transpose(x5, (1, 3, 0, 2, 4))
    p = p6.reshape(TN, P).astype(jnp.bfloat16)
    ze = jnp.dot(p, w1_ref[...],
                 preferred_element_type=jnp.float32) + encb_ref[...]
    z = jnp.dot(ze.astype(jnp.bfloat16), prewt_ref[...],
                preferred_element_type=jnp.float32) + preb_ref[...]
    z_ref[0] = z.T
    mm = jnp.dot(z.astype(jnp.bfloat16), cbt_ref[...],
                 preferred_element_type=jnp.float32)
    rnorm = jnp.sum(z * z, axis=1, keepdims=True)
    dist = (rnorm + cnorm_ref[...]) - 2.0 * mm
    m = jnp.min(dist, axis=1, keepdims=True)
    iota = lax.broadcasted_iota(jnp.int32, dist.shape, 1)
    cand = jnp.where(dist == m, iota, K)
    tok = jnp.min(cand, axis=1, keepdims=True)
    onehot = (iota == tok).astype(jnp.bfloat16)
    g = jnp.dot(onehot, gt_ref[...],
                preferred_element_type=jnp.float32)
    zq_ref[0] = g[:, :E].T
    r5 = g[:, E:].reshape(HP, HP, C, DS, DS)
    r6 = jnp.transpose(r5, (2, 0, 3, 1, 4))
    rec_ref[0] = r6.reshape(C, H, W)


def _precompute_body(cb_ref, postwt_ref, wd2_ref, postb_ref, decb_ref,
                     cnorm_ref, gt_ref):
    wpd = jnp.dot(postwt_ref[...], wd2_ref[...],
                  preferred_element_type=jnp.float32,
                  precision=lax.Precision.HIGHEST)
    vb = jnp.dot(postb_ref[...], wd2_ref[...],
                 preferred_element_type=jnp.float32,
                 precision=lax.Precision.HIGHEST) + decb_ref[...]
    cb = cb_ref[...]
    cb2 = jnp.dot(cb, wpd, preferred_element_type=jnp.float32,
                  precision=lax.Precision.HIGHEST) + vb
    cnorm_ref[0, :] = jnp.sum(cb * cb, axis=1)
    gt_ref[:, :E] = cb.astype(jnp.bfloat16)
    gt_ref[:, E:] = cb2.astype(jnp.bfloat16)


def kernel(x, enc_W, enc_b, pre_W, pre_b, codebook, post_W, post_b, dec_W, dec_b):
    w1 = enc_W.reshape(ZC, P).T.astype(jnp.bfloat16)
    prewt = pre_W.T.astype(jnp.bfloat16)
    cbt = codebook.T.astype(jnp.bfloat16)
    wd2 = dec_W[:, :, ::-1, ::-1].transpose(1, 0, 2, 3).reshape(ZC, P)
    postwt = post_W.T
    encb2 = enc_b.reshape(1, ZC)
    preb2 = pre_b.reshape(1, E)
    postb2 = post_b.reshape(1, ZC)
    decb2 = jnp.repeat(dec_b, DS * DS).reshape(1, P)

    cnorm, gtable = pl.pallas_call(
        _precompute_body,
        out_shape=(
            jax.ShapeDtypeStruct((1, K), jnp.float32),
            jax.ShapeDtypeStruct((K, G), jnp.bfloat16),
        ),
    )(codebook, postwt, wd2, postb2, decb2)

    zt, zqt, rec = pl.pallas_call(
        _main_body,
        grid=(B,),
        in_specs=[
            pl.BlockSpec((1, C, H, W), lambda i: (i, 0, 0, 0)),
            pl.BlockSpec((P, ZC), lambda i: (0, 0)),
            pl.BlockSpec((1, ZC), lambda i: (0, 0)),
            pl.BlockSpec((ZC, E), lambda i: (0, 0)),
            pl.BlockSpec((1, E), lambda i: (0, 0)),
            pl.BlockSpec((E, K), lambda i: (0, 0)),
            pl.BlockSpec((1, K), lambda i: (0, 0)),
            pl.BlockSpec((K, G), lambda i: (0, 0)),
        ],
        out_specs=(
            pl.BlockSpec((1, E, TN), lambda i: (i, 0, 0)),
            pl.BlockSpec((1, E, TN), lambda i: (i, 0, 0)),
            pl.BlockSpec((1, C, H, W), lambda i: (i, 0, 0, 0)),
        ),
        out_shape=(
            jax.ShapeDtypeStruct((B, E, TN), jnp.float32),
            jax.ShapeDtypeStruct((B, E, TN), jnp.float32),
            jax.ShapeDtypeStruct((B, C, H, W), jnp.float32),
        ),
    )(x, w1, encb2, prewt, preb2, cbt, cnorm, gtable)

    z = zt.reshape(B, E, HP, HP)
    z_q = zqt.reshape(B, E, HP, HP)
    return (z, z_q, rec)


# speedup vs baseline: 2.6363x; 1.0421x over previous
import functools

import jax
import jax.numpy as jnp
from jax import lax
from jax.experimental import pallas as pl
from jax.experimental.pallas import tpu as pltpu

B, C, H, W = 8, 3, 224, 224
ZC = 256
E = 256
K = 1024
DS = 8
HP = H // DS
N = B * HP * HP
P = C * DS * DS
G = E + P
TN = HP * HP


def _main_body(x_ref, w1_ref, encb_ref, prewt_ref, preb_ref, cbt_ref,
               cb_ref, postwt_ref, wd2_ref, postb_ref, decb_ref,
               z_ref, zq_ref, rec_ref, cnorm_v, gt_v):
    @pl.when(pl.program_id(0) == 0)
    def _precompute():
        wpd = jnp.dot(postwt_ref[...], wd2_ref[...],
                      preferred_element_type=jnp.float32,
                      precision=lax.Precision.HIGHEST)
        vb = jnp.dot(postb_ref[...], wd2_ref[...],
                     preferred_element_type=jnp.float32,
                     precision=lax.Precision.HIGHEST) + decb_ref[...]
        cb = cb_ref[...]
        cb2 = jnp.dot(cb, wpd, preferred_element_type=jnp.float32,
                      precision=lax.Precision.HIGHEST) + vb
        cnorm_v[0, :] = jnp.sum(cb * cb, axis=1)
        gt_v[:, :E] = cb.astype(jnp.bfloat16)
        gt_v[:, E:] = cb2.astype(jnp.bfloat16)

    xb = x_ref[0]
    x5 = xb.reshape(C, HP, DS, HP, DS)
    p6 = jnp.transpose(x5, (1, 3, 0, 2, 4))
    p = p6.reshape(TN, P)
    ze = jnp.dot(p, w1_ref[...],
                 preferred_element_type=jnp.float32) + encb_ref[...]
    z = jnp.dot(ze.astype(jnp.bfloat16), prewt_ref[...],
                preferred_element_type=jnp.float32) + preb_ref[...]
    z_ref[0] = z.T
    mm = jnp.dot(z.astype(jnp.bfloat16), cbt_ref[...],
                 preferred_element_type=jnp.float32)
    rnorm = jnp.sum(z * z, axis=1, keepdims=True)
    dist = (rnorm + cnorm_v[...]) - 2.0 * mm
    m = jnp.min(dist, axis=1, keepdims=True)
    iota = lax.broadcasted_iota(jnp.int32, dist.shape, 1)
    cand = jnp.where(dist == m, iota, K)
    tok = jnp.min(cand, axis=1, keepdims=True)
    onehot = (iota == tok).astype(jnp.bfloat16)
    g = jnp.dot(onehot, gt_v[...],
                preferred_element_type=jnp.float32)
    zq_ref[0] = g[:, :E].T
    r5 = g[:, E:].reshape(HP, HP, C, DS, DS)
    r6 = jnp.transpose(r5, (2, 0, 3, 1, 4))
    rec_ref[0] = r6.reshape(C, H, W)


def kernel(x, enc_W, enc_b, pre_W, pre_b, codebook, post_W, post_b, dec_W, dec_b):
    x16 = x.astype(jnp.bfloat16)
    w1 = enc_W.reshape(ZC, P).T.astype(jnp.bfloat16)
    prewt = pre_W.T.astype(jnp.bfloat16)
    cbt = codebook.T.astype(jnp.bfloat16)
    wd2 = dec_W[:, :, ::-1, ::-1].transpose(1, 0, 2, 3).reshape(ZC, P)
    postwt = post_W.T
    encb2 = enc_b.reshape(1, ZC)
    preb2 = pre_b.reshape(1, E)
    postb2 = post_b.reshape(1, ZC)
    decb2 = jnp.repeat(dec_b, DS * DS).reshape(1, P)

    zt, zqt, rec = pl.pallas_call(
        _main_body,
        grid=(B,),
        in_specs=[
            pl.BlockSpec((1, C, H, W), lambda i: (i, 0, 0, 0)),
            pl.BlockSpec((P, ZC), lambda i: (0, 0)),
            pl.BlockSpec((1, ZC), lambda i: (0, 0)),
            pl.BlockSpec((ZC, E), lambda i: (0, 0)),
            pl.BlockSpec((1, E), lambda i: (0, 0)),
            pl.BlockSpec((E, K), lambda i: (0, 0)),
            pl.BlockSpec((K, E), lambda i: (0, 0)),
            pl.BlockSpec((E, ZC), lambda i: (0, 0)),
            pl.BlockSpec((ZC, P), lambda i: (0, 0)),
            pl.BlockSpec((1, ZC), lambda i: (0, 0)),
            pl.BlockSpec((1, P), lambda i: (0, 0)),
        ],
        out_specs=(
            pl.BlockSpec((1, E, TN), lambda i: (i, 0, 0)),
            pl.BlockSpec((1, E, TN), lambda i: (i, 0, 0)),
            pl.BlockSpec((1, C, H, W), lambda i: (i, 0, 0, 0)),
        ),
        out_shape=(
            jax.ShapeDtypeStruct((B, E, TN), jnp.float32),
            jax.ShapeDtypeStruct((B, E, TN), jnp.float32),
            jax.ShapeDtypeStruct((B, C, H, W), jnp.float32),
        ),
        scratch_shapes=[
            pltpu.VMEM((1, K), jnp.float32),
            pltpu.VMEM((K, G), jnp.bfloat16),
        ],
    )(x16, w1, encb2, prewt, preb2, cbt, codebook, postwt, wd2, postb2, decb2)

    z = zt.reshape(B, E, HP, HP)
    z_q = zqt.reshape(B, E, HP, HP)
    return (z, z_q, rec)
